# baseline (speedup 1.0000x reference)
import functools

import jax
import jax.numpy as jnp
from jax.experimental import pallas as pl
from jax.experimental.pallas import tpu as pltpu

_BM = 256
_F32 = jnp.float32
_HI = jax.lax.Precision.HIGHEST


def _split(hs):
    hi = hs.astype(jnp.bfloat16)
    lo = (hs - hi.astype(_F32)).astype(jnp.bfloat16)
    return hi, lo


def _prescale_body(h_ref, dc_ref, hi_ref, lo_ref):
    hs = h_ref[...] * dc_ref[...]
    hi, lo = _split(hs)
    hi_ref[...] = hi
    lo_ref[...] = lo


def _prescale(h, dcol, bn=None):
    NP, d = h.shape
    body = _prescale_body if bn is None else _bnprescale_body
    args = (h,) if bn is None else (h, bn[0].reshape(1, -1), bn[1].reshape(1, -1))
    extra_specs = [] if bn is None else [
        pl.BlockSpec((1, d), lambda k: (0, 0)),
        pl.BlockSpec((1, d), lambda k: (0, 0)),
    ]
    return pl.pallas_call(
        body,
        grid=(NP // 2048,),
        in_specs=[pl.BlockSpec((2048, d), lambda k: (k, 0))] + extra_specs
        + [pl.BlockSpec((2048, 1), lambda k: (k, 0))],
        out_specs=[
            pl.BlockSpec((2048, d), lambda k: (k, 0)),
            pl.BlockSpec((2048, d), lambda k: (k, 0)),
        ],
        out_shape=[
            jax.ShapeDtypeStruct((NP, d), jnp.bfloat16),
            jax.ShapeDtypeStruct((NP, d), jnp.bfloat16),
        ],
    )(*args, dcol)


def _bnprescale_body(z_ref, s_ref, t_ref, dc_ref, hi_ref, lo_ref):
    h = jnp.maximum(z_ref[...] * s_ref[...] + t_ref[...], 0.0)
    hs = h * dc_ref[...]
    hi, lo = _split(hs)
    hi_ref[...] = hi
    lo_ref[...] = lo


def _layer_body(m_ref, hi_ref, lo_ref, dr_ref, w_ref, b_ref, *out_refs,
                mode):
    i = pl.program_id(0)
    agg = jnp.dot(m_ref[...], hi_ref[...], preferred_element_type=_F32)
    agg += jnp.dot(m_ref[...], lo_ref[...], preferred_element_type=_F32)
    rows = pl.ds(i * _BM, _BM)
    agg += hi_ref[rows, :].astype(_F32) + lo_ref[rows, :].astype(_F32)
    agg *= dr_ref[...]
    z = jnp.dot(agg, w_ref[...], preferred_element_type=_F32, precision=_HI)
    z += b_ref[...]
    if mode == "z":
        out_refs[0][...] = z
    else:
        hs = jnp.maximum(z, 0.0) * dr_ref[...]
        hi, lo = _split(hs)
        out_refs[0][...] = hi
        out_refs[1][...] = lo


def _layer(M, hi, lo, dinv_col, W, b, mode):
    NP = M.shape[0]
    d_in = hi.shape[1]
    d_out = W.shape[1]
    if mode == "z":
        out_specs = [pl.BlockSpec((_BM, d_out), lambda i: (i, 0))]
        out_shape = [jax.ShapeDtypeStruct((NP, d_out), _F32)]
    else:
        out_specs = [pl.BlockSpec((_BM, d_out), lambda i: (i, 0)),
                     pl.BlockSpec((_BM, d_out), lambda i: (i, 0))]
        out_shape = [jax.ShapeDtypeStruct((NP, d_out), jnp.bfloat16),
                     jax.ShapeDtypeStruct((NP, d_out), jnp.bfloat16)]
    outs = pl.pallas_call(
        functools.partial(_layer_body, mode=mode),
        grid=(NP // _BM,),
        in_specs=[
            pl.BlockSpec((_BM, NP), lambda i: (i, 0)),
            pl.BlockSpec((NP, d_in), lambda i: (0, 0)),
            pl.BlockSpec((NP, d_in), lambda i: (0, 0)),
            pl.BlockSpec((_BM, 1), lambda i: (i, 0)),
            pl.BlockSpec((d_in, d_out), lambda i: (0, 0)),
            pl.BlockSpec((1, d_out), lambda i: (0, 0)),
        ],
        out_specs=out_specs,
        out_shape=out_shape,
        compiler_params=pltpu.CompilerParams(
            dimension_semantics=("arbitrary",),
        ),
    )(M, hi, lo, dinv_col, W, b.reshape(1, -1))
    return outs


def _head_body(z_ref, s_ref, t_ref, wp_ref, bp_ref, vit_ref, wlv_ref, wlh_ref,
               bl_ref, o_ref, acc_ref, *, nk):
    k = pl.program_id(0)

    @pl.when(k == 0)
    def _():
        acc_ref[...] = jnp.zeros_like(acc_ref)

    hb = jnp.maximum(z_ref[...] * s_ref[...] + t_ref[...], 0.0)
    acc_ref[...] += jnp.dot(wp_ref[...], hb, preferred_element_type=_F32,
                            precision=_HI)

    @pl.when(k == nk - 1)
    def _():
        hp = acc_ref[...] + bp_ref[...]
        out = jnp.dot(vit_ref[...], wlv_ref[...], preferred_element_type=_F32,
                      precision=_HI)
        out += jnp.dot(hp, wlh_ref[...], preferred_element_type=_F32,
                       precision=_HI)
        o_ref[...] = out + bl_ref[...]


def _head(z4, scale, shift, Wp, bp, vit, Wlv, Wlh, blin, BK=2048):
    NP, d = z4.shape
    TP = Wp.shape[0]
    d_out = Wlh.shape[1]
    nk = NP // BK
    return pl.pallas_call(
        functools.partial(_head_body, nk=nk),
        grid=(nk,),
        in_specs=[
            pl.BlockSpec((BK, d), lambda k: (k, 0)),
            pl.BlockSpec((1, d), lambda k: (0, 0)),
            pl.BlockSpec((1, d), lambda k: (0, 0)),
            pl.BlockSpec((TP, BK), lambda k: (0, k)),
            pl.BlockSpec((TP, 1), lambda k: (0, 0)),
            pl.BlockSpec((TP, vit.shape[1]), lambda k: (0, 0)),
            pl.BlockSpec((vit.shape[1], d_out), lambda k: (0, 0)),
            pl.BlockSpec((d, d_out), lambda k: (0, 0)),
            pl.BlockSpec((1, d_out), lambda k: (0, 0)),
        ],
        out_specs=pl.BlockSpec((TP, d_out), lambda k: (0, 0)),
        out_shape=jax.ShapeDtypeStruct((TP, d_out), _F32),
        scratch_shapes=[pltpu.VMEM((TP, d), _F32)],
        compiler_params=pltpu.CompilerParams(
            dimension_semantics=("arbitrary",),
        ),
    )(z4, scale.reshape(1, -1), shift.reshape(1, -1), Wp, bp.reshape(-1, 1),
      vit, Wlv, Wlh, blin.reshape(1, -1))


def _rowsum_body(m_ref, o_ref, *, nc):
    c = pl.program_id(1)

    @pl.when(c == 0)
    def _():
        o_ref[...] = jnp.zeros_like(o_ref)

    o_ref[...] += jnp.sum(m_ref[...].astype(_F32), axis=1, keepdims=True)

    @pl.when(c == nc - 1)
    def _():
        o_ref[...] = jax.lax.rsqrt(o_ref[...] + 1.0)


def _dinv_from_M(M, BR=2048, BC=2048):
    NP = M.shape[0]
    nc = NP // BC
    return pl.pallas_call(
        functools.partial(_rowsum_body, nc=nc),
        grid=(NP // BR, nc),
        in_specs=[pl.BlockSpec((BR, BC), lambda r, c: (r, c))],
        out_specs=pl.BlockSpec((BR, 1), lambda r, c: (r, 0)),
        out_shape=jax.ShapeDtypeStruct((NP, 1), _F32),
        compiler_params=pltpu.CompilerParams(
            dimension_semantics=("arbitrary", "arbitrary"),
        ),
    )(M)


def _bn_coeffs(z, N, g, be):
    zn = z[:N]
    m = zn.mean(axis=0)
    v = zn.var(axis=0)
    scale = g * jax.lax.rsqrt(v + 1e-5)
    shift = be - m * scale
    return scale, shift


def kernel(features_list, edges_list, vit_output, W1, b1, W2, b2, g1, be1,
           W3, b3, W4, b4, g2, be2, Wproj, bproj, Wlin, blin):
    x = features_list[0]
    ei = edges_list[0]
    vit = vit_output[0]
    N = x.shape[0]
    NP = ((N + 2047) // 2048) * 2048
    T = Wproj.shape[0]
    TP = ((T + 255) // 256) * 256

    src = ei[0]
    dst = ei[1]
    M = jnp.zeros((NP, NP), jnp.bfloat16).at[dst, src].add(
        jnp.ones(src.shape, jnp.bfloat16))

    dinvp = _dinv_from_M(M)

    xp = jnp.zeros((NP, x.shape[1]), _F32).at[:N].set(x)
    hi0, lo0 = _prescale(xp, dinvp)

    hi1, lo1 = _layer(M, hi0, lo0, dinvp, W1, b1, mode="hs")
    z2, = _layer(M, hi1, lo1, dinvp, W2, b2, mode="z")
    s1, t1 = _bn_coeffs(z2, N, g1, be1)
    hi2, lo2 = _prescale(z2, dinvp, bn=(s1, t1))
    hi3, lo3 = _layer(M, hi2, lo2, dinvp, W3, b3, mode="hs")
    z4, = _layer(M, hi3, lo3, dinvp, W4, b4, mode="z")
    s2, t2 = _bn_coeffs(z4, N, g2, be2)

    Wp = jnp.zeros((TP, NP), _F32).at[:T, :N].set(Wproj)
    bp = jnp.zeros((TP,), _F32).at[:T].set(bproj)
    vitp = jnp.zeros((TP, vit.shape[1]), _F32).at[:T].set(vit)
    Wlv = Wlin[: vit.shape[1]]
    Wlh = Wlin[vit.shape[1]:]

    out = _head(z4, s2, t2, Wp, bp, vitp, Wlv, Wlh, blin)
    return out[:T][None]


# speedup vs baseline: 1.7390x; 1.7390x over previous
import functools

import jax
import jax.numpy as jnp
from jax.experimental import pallas as pl
from jax.experimental.pallas import tpu as pltpu

_BM = 256
_F32 = jnp.float32
_HI = jax.lax.Precision.HIGHEST


def _split(hs):
    hi = hs.astype(jnp.bfloat16)
    lo = (hs - hi.astype(_F32)).astype(jnp.bfloat16)
    return hi, lo


def _prescale_body(h_ref, dc_ref, hi_ref, lo_ref):
    hs = h_ref[...] * dc_ref[...]
    hi, lo = _split(hs)
    hi_ref[...] = hi
    lo_ref[...] = lo


def _prescale(h, dcol, bn=None):
    NP, d = h.shape
    body = _prescale_body if bn is None else _bnprescale_body
    args = (h,) if bn is None else (h, bn[0].reshape(1, -1), bn[1].reshape(1, -1))
    extra_specs = [] if bn is None else [
        pl.BlockSpec((1, d), lambda k: (0, 0)),
        pl.BlockSpec((1, d), lambda k: (0, 0)),
    ]
    return pl.pallas_call(
        body,
        grid=(NP // 2048,),
        in_specs=[pl.BlockSpec((2048, d), lambda k: (k, 0))] + extra_specs
        + [pl.BlockSpec((2048, 1), lambda k: (k, 0))],
        out_specs=[
            pl.BlockSpec((2048, d), lambda k: (k, 0)),
            pl.BlockSpec((2048, d), lambda k: (k, 0)),
        ],
        out_shape=[
            jax.ShapeDtypeStruct((NP, d), jnp.bfloat16),
            jax.ShapeDtypeStruct((NP, d), jnp.bfloat16),
        ],
    )(*args, dcol)


def _bnprescale_body(z_ref, s_ref, t_ref, dc_ref, hi_ref, lo_ref):
    h = jnp.maximum(z_ref[...] * s_ref[...] + t_ref[...], 0.0)
    hs = h * dc_ref[...]
    hi, lo = _split(hs)
    hi_ref[...] = hi
    lo_ref[...] = lo


def _layer_body(m_ref, hi_ref, lo_ref, dr_ref, w_ref, b_ref, *out_refs,
                mode):
    i = pl.program_id(0)
    agg = jnp.dot(m_ref[...], hi_ref[...], preferred_element_type=_F32)
    agg += jnp.dot(m_ref[...], lo_ref[...], preferred_element_type=_F32)
    rows = pl.ds(i * _BM, _BM)
    agg += hi_ref[rows, :].astype(_F32) + lo_ref[rows, :].astype(_F32)
    agg *= dr_ref[...]
    z = jnp.dot(agg, w_ref[...], preferred_element_type=_F32, precision=_HI)
    z += b_ref[...]
    if mode == "z":
        out_refs[0][...] = z
    else:
        hs = jnp.maximum(z, 0.0) * dr_ref[...]
        hi, lo = _split(hs)
        out_refs[0][...] = hi
        out_refs[1][...] = lo


def _layer(M, hi, lo, dinv_col, W, b, mode):
    NP = M.shape[0]
    d_in = hi.shape[1]
    d_out = W.shape[1]
    if mode == "z":
        out_specs = [pl.BlockSpec((_BM, d_out), lambda i: (i, 0))]
        out_shape = [jax.ShapeDtypeStruct((NP, d_out), _F32)]
    else:
        out_specs = [pl.BlockSpec((_BM, d_out), lambda i: (i, 0)),
                     pl.BlockSpec((_BM, d_out), lambda i: (i, 0))]
        out_shape = [jax.ShapeDtypeStruct((NP, d_out), jnp.bfloat16),
                     jax.ShapeDtypeStruct((NP, d_out), jnp.bfloat16)]
    outs = pl.pallas_call(
        functools.partial(_layer_body, mode=mode),
        grid=(NP // _BM,),
        in_specs=[
            pl.BlockSpec((_BM, NP), lambda i: (i, 0)),
            pl.BlockSpec((NP, d_in), lambda i: (0, 0)),
            pl.BlockSpec((NP, d_in), lambda i: (0, 0)),
            pl.BlockSpec((_BM, 1), lambda i: (i, 0)),
            pl.BlockSpec((d_in, d_out), lambda i: (0, 0)),
            pl.BlockSpec((1, d_out), lambda i: (0, 0)),
        ],
        out_specs=out_specs,
        out_shape=out_shape,
        compiler_params=pltpu.CompilerParams(
            dimension_semantics=("arbitrary",),
        ),
    )(M, hi, lo, dinv_col, W, b.reshape(1, -1))
    return outs


def _head_body(z_ref, s_ref, t_ref, wp_ref, bp_ref, vit_ref, wlv_ref, wlh_ref,
               bl_ref, o_ref, acc_ref, *, nk):
    k = pl.program_id(0)

    @pl.when(k == 0)
    def _():
        acc_ref[...] = jnp.zeros_like(acc_ref)

    hb = jnp.maximum(z_ref[...] * s_ref[...] + t_ref[...], 0.0)
    acc_ref[...] += jnp.dot(wp_ref[...], hb, preferred_element_type=_F32,
                            precision=_HI)

    @pl.when(k == nk - 1)
    def _():
        hp = acc_ref[...] + bp_ref[...]
        out = jnp.dot(vit_ref[...], wlv_ref[...], preferred_element_type=_F32,
                      precision=_HI)
        out += jnp.dot(hp, wlh_ref[...], preferred_element_type=_F32,
                       precision=_HI)
        o_ref[...] = out + bl_ref[...]


def _head(z4, scale, shift, Wp, bp, vit, Wlv, Wlh, blin, BK=2048):
    NP, d = z4.shape
    TP = Wp.shape[0]
    d_out = Wlh.shape[1]
    nk = NP // BK
    return pl.pallas_call(
        functools.partial(_head_body, nk=nk),
        grid=(nk,),
        in_specs=[
            pl.BlockSpec((BK, d), lambda k: (k, 0)),
            pl.BlockSpec((1, d), lambda k: (0, 0)),
            pl.BlockSpec((1, d), lambda k: (0, 0)),
            pl.BlockSpec((TP, BK), lambda k: (0, k)),
            pl.BlockSpec((TP, 1), lambda k: (0, 0)),
            pl.BlockSpec((TP, vit.shape[1]), lambda k: (0, 0)),
            pl.BlockSpec((vit.shape[1], d_out), lambda k: (0, 0)),
            pl.BlockSpec((d, d_out), lambda k: (0, 0)),
            pl.BlockSpec((1, d_out), lambda k: (0, 0)),
        ],
        out_specs=pl.BlockSpec((TP, d_out), lambda k: (0, 0)),
        out_shape=jax.ShapeDtypeStruct((TP, d_out), _F32),
        scratch_shapes=[pltpu.VMEM((TP, d), _F32)],
        compiler_params=pltpu.CompilerParams(
            dimension_semantics=("arbitrary",),
        ),
    )(z4, scale.reshape(1, -1), shift.reshape(1, -1), Wp, bp.reshape(-1, 1),
      vit, Wlv, Wlh, blin.reshape(1, -1))


def _rowsum_body(m_ref, o_ref, *, nc):
    c = pl.program_id(1)

    @pl.when(c == 0)
    def _():
        o_ref[...] = jnp.zeros_like(o_ref)

    o_ref[...] += jnp.sum(m_ref[...].astype(_F32), axis=1, keepdims=True)

    @pl.when(c == nc - 1)
    def _():
        o_ref[...] = jax.lax.rsqrt(o_ref[...] + 1.0)


def _dinv_from_M(M, BR=2048, BC=2048):
    NP = M.shape[0]
    nc = NP // BC
    return pl.pallas_call(
        functools.partial(_rowsum_body, nc=nc),
        grid=(NP // BR, nc),
        in_specs=[pl.BlockSpec((BR, BC), lambda r, c: (r, c))],
        out_specs=pl.BlockSpec((BR, 1), lambda r, c: (r, 0)),
        out_shape=jax.ShapeDtypeStruct((NP, 1), _F32),
        compiler_params=pltpu.CompilerParams(
            dimension_semantics=("arbitrary", "arbitrary"),
        ),
    )(M)


def _bn_coeffs(z, N, g, be):
    zn = z[:N]
    m = zn.mean(axis=0)
    v = zn.var(axis=0)
    scale = g * jax.lax.rsqrt(v + 1e-5)
    shift = be - m * scale
    return scale, shift


def kernel(features_list, edges_list, vit_output, W1, b1, W2, b2, g1, be1,
           W3, b3, W4, b4, g2, be2, Wproj, bproj, Wlin, blin):
    x = features_list[0]
    ei = edges_list[0]
    vit = vit_output[0]
    N = x.shape[0]
    NP = ((N + 2047) // 2048) * 2048
    T = Wproj.shape[0]
    TP = ((T + 255) // 256) * 256

    src = ei[0]
    dst = ei[1]
    M = jnp.zeros((NP, NP), _F32).at[dst, src].add(1.0).astype(jnp.bfloat16)

    dinvp = _dinv_from_M(M)

    xp = jnp.zeros((NP, x.shape[1]), _F32).at[:N].set(x)
    hi0, lo0 = _prescale(xp, dinvp)

    hi1, lo1 = _layer(M, hi0, lo0, dinvp, W1, b1, mode="hs")
    z2, = _layer(M, hi1, lo1, dinvp, W2, b2, mode="z")
    s1, t1 = _bn_coeffs(z2, N, g1, be1)
    hi2, lo2 = _prescale(z2, dinvp, bn=(s1, t1))
    hi3, lo3 = _layer(M, hi2, lo2, dinvp, W3, b3, mode="hs")
    z4, = _layer(M, hi3, lo3, dinvp, W4, b4, mode="z")
    s2, t2 = _bn_coeffs(z4, N, g2, be2)

    Wp = jnp.zeros((TP, NP), _F32).at[:T, :N].set(Wproj)
    bp = jnp.zeros((TP,), _F32).at[:T].set(bproj)
    vitp = jnp.zeros((TP, vit.shape[1]), _F32).at[:T].set(vit)
    Wlv = Wlin[: vit.shape[1]]
    Wlh = Wlin[vit.shape[1]:]

    out = _head(z4, s2, t2, Wp, bp, vitp, Wlv, Wlh, blin)
    return out[:T][None]


# speedup vs baseline: 1.7935x; 1.0313x over previous
import functools

import jax
import jax.numpy as jnp
from jax.experimental import pallas as pl
from jax.experimental.pallas import tpu as pltpu

_BM = 256
_F32 = jnp.float32


def _dot3(a, w_hi, w_lo):
    a_hi = a.astype(jnp.bfloat16)
    a_lo = (a - a_hi.astype(_F32)).astype(jnp.bfloat16)
    out = jnp.dot(a_hi, w_hi, preferred_element_type=_F32)
    out += jnp.dot(a_hi, w_lo, preferred_element_type=_F32)
    out += jnp.dot(a_lo, w_hi, preferred_element_type=_F32)
    return out


def _split(hs):
    hi = hs.astype(jnp.bfloat16)
    lo = (hs - hi.astype(_F32)).astype(jnp.bfloat16)
    return hi, lo


def _prescale_body(h_ref, dc_ref, hi_ref, lo_ref):
    hs = h_ref[...] * dc_ref[...]
    hi, lo = _split(hs)
    hi_ref[...] = hi
    lo_ref[...] = lo


def _prescale(h, dcol, bn=None):
    NP, d = h.shape
    body = _prescale_body if bn is None else _bnprescale_body
    args = (h,) if bn is None else (h, bn[0].reshape(1, -1), bn[1].reshape(1, -1))
    extra_specs = [] if bn is None else [
        pl.BlockSpec((1, d), lambda k: (0, 0)),
        pl.BlockSpec((1, d), lambda k: (0, 0)),
    ]
    return pl.pallas_call(
        body,
        grid=(NP // 2048,),
        in_specs=[pl.BlockSpec((2048, d), lambda k: (k, 0))] + extra_specs
        + [pl.BlockSpec((2048, 1), lambda k: (k, 0))],
        out_specs=[
            pl.BlockSpec((2048, d), lambda k: (k, 0)),
            pl.BlockSpec((2048, d), lambda k: (k, 0)),
        ],
        out_shape=[
            jax.ShapeDtypeStruct((NP, d), jnp.bfloat16),
            jax.ShapeDtypeStruct((NP, d), jnp.bfloat16),
        ],
    )(*args, dcol)


def _bnprescale_body(z_ref, s_ref, t_ref, dc_ref, hi_ref, lo_ref):
    h = jnp.maximum(z_ref[...] * s_ref[...] + t_ref[...], 0.0)
    hs = h * dc_ref[...]
    hi, lo = _split(hs)
    hi_ref[...] = hi
    lo_ref[...] = lo


def _layer_body(m_ref, hi_ref, lo_ref, dr_ref, whi_ref, wlo_ref, b_ref,
                *out_refs, mode):
    i = pl.program_id(0)
    m = m_ref[...].astype(jnp.bfloat16)
    agg = jnp.dot(m, hi_ref[...], preferred_element_type=_F32)
    agg += jnp.dot(m, lo_ref[...], preferred_element_type=_F32)
    rows = pl.ds(i * _BM, _BM)
    agg += hi_ref[rows, :].astype(_F32) + lo_ref[rows, :].astype(_F32)
    agg *= dr_ref[...]
    z = _dot3(agg, whi_ref[...], wlo_ref[...]) + b_ref[...]
    if mode == "z":
        out_refs[0][...] = z
    else:
        hs = jnp.maximum(z, 0.0) * dr_ref[...]
        hi, lo = _split(hs)
        out_refs[0][...] = hi
        out_refs[1][...] = lo


def _layer(M, hi, lo, dinv_col, W, b, mode):
    NP = M.shape[0]
    d_in = hi.shape[1]
    d_out = W.shape[1]
    if mode == "z":
        out_specs = [pl.BlockSpec((_BM, d_out), lambda i: (i, 0))]
        out_shape = [jax.ShapeDtypeStruct((NP, d_out), _F32)]
    else:
        out_specs = [pl.BlockSpec((_BM, d_out), lambda i: (i, 0)),
                     pl.BlockSpec((_BM, d_out), lambda i: (i, 0))]
        out_shape = [jax.ShapeDtypeStruct((NP, d_out), jnp.bfloat16),
                     jax.ShapeDtypeStruct((NP, d_out), jnp.bfloat16)]
    outs = pl.pallas_call(
        functools.partial(_layer_body, mode=mode),
        grid=(NP // _BM,),
        in_specs=[
            pl.BlockSpec((_BM, NP), lambda i: (i, 0)),
            pl.BlockSpec((NP, d_in), lambda i: (0, 0)),
            pl.BlockSpec((NP, d_in), lambda i: (0, 0)),
            pl.BlockSpec((_BM, 1), lambda i: (i, 0)),
            pl.BlockSpec((d_in, d_out), lambda i: (0, 0)),
            pl.BlockSpec((d_in, d_out), lambda i: (0, 0)),
            pl.BlockSpec((1, d_out), lambda i: (0, 0)),
        ],
        out_specs=out_specs,
        out_shape=out_shape,
        compiler_params=pltpu.CompilerParams(
            dimension_semantics=("arbitrary",),
        ),
    )(M, hi, lo, dinv_col, *_wsplit(W), b.reshape(1, -1))
    return outs


def _wsplit(W):
    w_hi = W.astype(jnp.bfloat16)
    w_lo = (W - w_hi.astype(_F32)).astype(jnp.bfloat16)
    return w_hi, w_lo


def _head_body(z_ref, s_ref, t_ref, wp_ref, bp_ref, vit_ref, wlvhi_ref,
               wlvlo_ref, wlhhi_ref, wlhlo_ref, bl_ref, o_ref, acc_ref, *, nk):
    k = pl.program_id(0)

    @pl.when(k == 0)
    def _():
        acc_ref[...] = jnp.zeros_like(acc_ref)

    hb = jnp.maximum(z_ref[...] * s_ref[...] + t_ref[...], 0.0)
    hb_hi = hb.astype(jnp.bfloat16)
    hb_lo = (hb - hb_hi.astype(_F32)).astype(jnp.bfloat16)
    wp = wp_ref[...]
    wp_hi = wp.astype(jnp.bfloat16)
    wp_lo = (wp - wp_hi.astype(_F32)).astype(jnp.bfloat16)
    acc = jnp.dot(wp_hi, hb_hi, preferred_element_type=_F32)
    acc += jnp.dot(wp_hi, hb_lo, preferred_element_type=_F32)
    acc += jnp.dot(wp_lo, hb_hi, preferred_element_type=_F32)
    acc_ref[...] += acc

    @pl.when(k == nk - 1)
    def _():
        hp = acc_ref[...] + bp_ref[...]
        out = _dot3(vit_ref[...], wlvhi_ref[...], wlvlo_ref[...])
        out += _dot3(hp, wlhhi_ref[...], wlhlo_ref[...])
        o_ref[...] = out + bl_ref[...]


def _head(z4, scale, shift, Wp, bp, vit, Wlv, Wlh, blin, BK=2048):
    NP, d = z4.shape
    TP = Wp.shape[0]
    d_out = Wlh.shape[1]
    nk = NP // BK
    return pl.pallas_call(
        functools.partial(_head_body, nk=nk),
        grid=(nk,),
        in_specs=[
            pl.BlockSpec((BK, d), lambda k: (k, 0)),
            pl.BlockSpec((1, d), lambda k: (0, 0)),
            pl.BlockSpec((1, d), lambda k: (0, 0)),
            pl.BlockSpec((TP, BK), lambda k: (0, k)),
            pl.BlockSpec((TP, 1), lambda k: (0, 0)),
            pl.BlockSpec((TP, vit.shape[1]), lambda k: (0, 0)),
            pl.BlockSpec((vit.shape[1], d_out), lambda k: (0, 0)),
            pl.BlockSpec((vit.shape[1], d_out), lambda k: (0, 0)),
            pl.BlockSpec((d, d_out), lambda k: (0, 0)),
            pl.BlockSpec((d, d_out), lambda k: (0, 0)),
            pl.BlockSpec((1, d_out), lambda k: (0, 0)),
        ],
        out_specs=pl.BlockSpec((TP, d_out), lambda k: (0, 0)),
        out_shape=jax.ShapeDtypeStruct((TP, d_out), _F32),
        scratch_shapes=[pltpu.VMEM((TP, d), _F32)],
        compiler_params=pltpu.CompilerParams(
            dimension_semantics=("arbitrary",),
        ),
    )(z4, scale.reshape(1, -1), shift.reshape(1, -1), Wp, bp.reshape(-1, 1),
      vit, *_wsplit(Wlv), *_wsplit(Wlh), blin.reshape(1, -1))


def _rowsum_body(m_ref, o_ref, *, nc):
    c = pl.program_id(1)

    @pl.when(c == 0)
    def _():
        o_ref[...] = jnp.zeros_like(o_ref)

    o_ref[...] += jnp.sum(m_ref[...].astype(_F32), axis=1, keepdims=True)

    @pl.when(c == nc - 1)
    def _():
        o_ref[...] = jax.lax.rsqrt(o_ref[...] + 1.0)


def _dinv_from_M(M, BR=2048, BC=2048):
    NP = M.shape[0]
    nc = NP // BC
    return pl.pallas_call(
        functools.partial(_rowsum_body, nc=nc),
        grid=(NP // BR, nc),
        in_specs=[pl.BlockSpec((BR, BC), lambda r, c: (r, c))],
        out_specs=pl.BlockSpec((BR, 1), lambda r, c: (r, 0)),
        out_shape=jax.ShapeDtypeStruct((NP, 1), _F32),
        compiler_params=pltpu.CompilerParams(
            dimension_semantics=("arbitrary", "arbitrary"),
        ),
    )(M)


def _bn_coeffs(z, N, g, be):
    zn = z[:N]
    m = zn.mean(axis=0)
    v = zn.var(axis=0)
    scale = g * jax.lax.rsqrt(v + 1e-5)
    shift = be - m * scale
    return scale, shift


def kernel(features_list, edges_list, vit_output, W1, b1, W2, b2, g1, be1,
           W3, b3, W4, b4, g2, be2, Wproj, bproj, Wlin, blin):
    x = features_list[0]
    ei = edges_list[0]
    vit = vit_output[0]
    N = x.shape[0]
    NP = ((N + 2047) // 2048) * 2048
    T = Wproj.shape[0]
    TP = ((T + 255) // 256) * 256

    src = ei[0]
    dst = ei[1]
    M = jnp.zeros((NP, NP), _F32).at[dst, src].add(1.0)

    dinvp = _dinv_from_M(M)

    xp = jnp.zeros((NP, x.shape[1]), _F32).at[:N].set(x)
    hi0, lo0 = _prescale(xp, dinvp)

    hi1, lo1 = _layer(M, hi0, lo0, dinvp, W1, b1, mode="hs")
    z2, = _layer(M, hi1, lo1, dinvp, W2, b2, mode="z")
    s1, t1 = _bn_coeffs(z2, N, g1, be1)
    hi2, lo2 = _prescale(z2, dinvp, bn=(s1, t1))
    hi3, lo3 = _layer(M, hi2, lo2, dinvp, W3, b3, mode="hs")
    z4, = _layer(M, hi3, lo3, dinvp, W4, b4, mode="z")
    s2, t2 = _bn_coeffs(z4, N, g2, be2)

    Wp = jnp.zeros((TP, NP), _F32).at[:T, :N].set(Wproj)
    bp = jnp.zeros((TP,), _F32).at[:T].set(bproj)
    vitp = jnp.zeros((TP, vit.shape[1]), _F32).at[:T].set(vit)
    Wlv = Wlin[: vit.shape[1]]
    Wlh = Wlin[vit.shape[1]:]

    out = _head(z4, s2, t2, Wp, bp, vitp, Wlv, Wlh, blin)
    return out[:T][None]
